# Initial kernel scaffold; baseline (speedup 1.0000x reference)
import jax
import jax.numpy as jnp
from jax.experimental import pallas as pl


def kernel(idx, feats, edge_dict, sadj, epoch, W, b):
    raise NotImplementedError("write your pallas kernel here")


# speedup vs baseline: 1.5118x; 1.5118x over previous
import functools

import jax
import jax.numpy as jnp
from jax import lax
from jax.experimental import pallas as pl
from jax.experimental.pallas import tpu as pltpu
from jax.experimental.pallas import tpu_sc as plsc

N, K, D = 10000, 32, 128

NC, NS = 2, 16
NW = NC * NS
NPW = 320
NPAD = NW * NPW
G = 2
GK = G * K
NG = NPW // G
NBUF = 2
LANES = 16
DV = D // LANES

MM_BLOCK = 1000


def _mm_body(f_ref, w_ref, b_ref, o_ref):
    prod = lax.dot_general(f_ref[...], w_ref[...], (((1,), (1,)), ((), ())),
                           preferred_element_type=jnp.float32)
    o_ref[...] = jnp.maximum(prod + b_ref[...], 0.0)


_mm = pl.pallas_call(
    _mm_body,
    grid=(N // MM_BLOCK,),
    in_specs=[
        pl.BlockSpec((MM_BLOCK, D), lambda i: (i, 0)),
        pl.BlockSpec((D, D), lambda i: (0, 0)),
        pl.BlockSpec((1, D), lambda i: (0, 0)),
    ],
    out_specs=pl.BlockSpec((MM_BLOCK, D), lambda i: (i, 0)),
    out_shape=jax.ShapeDtypeStruct((N, D), jnp.float32),
)


def _sc_body(x_hbm, edge_hbm, out_hbm, idx_v, rows_v, out_v, sem0, sem1):
    sems = (sem0, sem1)
    wid = lax.axis_index("s") * NC + lax.axis_index("c")
    base = wid * NPW
    pltpu.sync_copy(edge_hbm.at[pl.ds(base * K, NPW * K)], idx_v)

    def _gather(g, slot):
        pltpu.async_copy(
            x_hbm.at[idx_v.at[pl.ds(g * GK, GK)]],
            rows_v.at[slot], sems[slot])

    for b in range(NBUF):
        _gather(b, b)

    @pl.loop(0, NG, step=NBUF)
    def _outer(g0):
        for b in range(NBUF):
            g = g0 + b
            pltpu.make_async_copy(
                x_hbm.at[idx_v.at[pl.ds(g * GK, GK)]],
                rows_v.at[b], sems[b]).wait()
            rows = rows_v.at[b]
            for gi in range(G):
                node = g * G + gi
                for dv in range(DV):
                    sl = pl.ds(dv * LANES, LANES)
                    acc = rows[gi * K, sl]
                    for k in range(1, K):
                        acc = acc + rows[gi * K + k, sl]
                    out_v[node, sl] = acc * (1.0 / K)
            gn = g + NBUF

            @pl.when(gn < NG)
            def _():
                _gather(gn, b)

    pltpu.sync_copy(out_v, out_hbm.at[pl.ds(base, NPW)])


_sc_gather_mean = pl.kernel(
    _sc_body,
    out_type=jax.ShapeDtypeStruct((NPAD, D), jnp.float32),
    mesh=plsc.VectorSubcoreMesh(core_axis_name="c", subcore_axis_name="s"),
    scratch_types=[
        pltpu.VMEM((NPW * K,), jnp.int32),
        pltpu.VMEM((NBUF, GK, D), jnp.float32),
        pltpu.VMEM((NPW, D), jnp.float32),
        pltpu.SemaphoreType.DMA,
        pltpu.SemaphoreType.DMA,
    ],
)


def kernel(idx, feats, edge_dict, sadj, epoch, W, b):
    x = _mm(feats, W, b.reshape(1, D))
    edge = jnp.concatenate(
        [edge_dict.astype(jnp.int32),
         jnp.zeros((NPAD - N, K), jnp.int32)]).reshape(-1)
    out = _sc_gather_mean(x, edge)
    return out[:N]


# speedup vs baseline: 1.5620x; 1.0332x over previous
import functools

import jax
import jax.numpy as jnp
from jax import lax
from jax.experimental import pallas as pl
from jax.experimental.pallas import tpu as pltpu
from jax.experimental.pallas import tpu_sc as plsc

N, K, D = 10000, 32, 128

NC, NS = 2, 16
NW = NC * NS
NPW = 320
NPAD = NW * NPW
G = 4
GK = G * K
NG = NPW // G
NBUF = 2
LANES = 16
DV = D // LANES

MM_BLOCK = 1000


def _mm_body(f_ref, w_ref, b_ref, o_ref):
    prod = lax.dot_general(f_ref[...], w_ref[...], (((1,), (1,)), ((), ())),
                           preferred_element_type=jnp.float32)
    o_ref[...] = jnp.maximum(prod + b_ref[...], 0.0)


_mm = pl.pallas_call(
    _mm_body,
    grid=(N // MM_BLOCK,),
    in_specs=[
        pl.BlockSpec((MM_BLOCK, D), lambda i: (i, 0)),
        pl.BlockSpec((D, D), lambda i: (0, 0)),
        pl.BlockSpec((1, D), lambda i: (0, 0)),
    ],
    out_specs=pl.BlockSpec((MM_BLOCK, D), lambda i: (i, 0)),
    out_shape=jax.ShapeDtypeStruct((N, D), jnp.float32),
)


def _sc_body(x_hbm, edge_hbm, out_hbm, idx_v, rows_v, out_v, sem0, sem1):
    sems = (sem0, sem1)
    wid = lax.axis_index("s") * NC + lax.axis_index("c")
    base = wid * NPW
    pltpu.sync_copy(edge_hbm.at[pl.ds(base * K, NPW * K)], idx_v)

    def _gather(g, slot):
        pltpu.async_copy(
            x_hbm.at[idx_v.at[pl.ds(g * GK, GK)]],
            rows_v.at[slot], sems[slot])

    for b in range(NBUF):
        _gather(b, b)

    @pl.loop(0, NG, step=NBUF)
    def _outer(g0):
        for b in range(NBUF):
            g = g0 + b
            pltpu.make_async_copy(
                x_hbm.at[idx_v.at[pl.ds(g * GK, GK)]],
                rows_v.at[b], sems[b]).wait()
            rows = rows_v.at[b]
            for gi in range(G):
                node = g * G + gi
                sls = [pl.ds(dv * LANES, LANES) for dv in range(DV)]
                accs = [rows[gi * K, sl] for sl in sls]
                for k in range(1, K):
                    r = gi * K + k
                    accs = [acc + rows[r, sl] for acc, sl in zip(accs, sls)]
                for dv in range(DV):
                    out_v[node, sls[dv]] = accs[dv] * (1.0 / K)
            gn = g + NBUF

            @pl.when(gn < NG)
            def _():
                _gather(gn, b)

    pltpu.sync_copy(out_v, out_hbm.at[pl.ds(base, NPW)])


_sc_gather_mean = pl.kernel(
    _sc_body,
    out_type=jax.ShapeDtypeStruct((NPAD, D), jnp.float32),
    mesh=plsc.VectorSubcoreMesh(core_axis_name="c", subcore_axis_name="s"),
    scratch_types=[
        pltpu.VMEM((NPW * K,), jnp.int32),
        pltpu.VMEM((NBUF, GK, D), jnp.float32),
        pltpu.VMEM((NPW, D), jnp.float32),
        pltpu.SemaphoreType.DMA,
        pltpu.SemaphoreType.DMA,
    ],
)


def kernel(idx, feats, edge_dict, sadj, epoch, W, b):
    x = _mm(feats, W, b.reshape(1, D))
    edge = jnp.concatenate(
        [edge_dict.astype(jnp.int32),
         jnp.zeros((NPAD - N, K), jnp.int32)]).reshape(-1)
    out = _sc_gather_mean(x, edge)
    return out[:N]
